# Initial kernel scaffold; baseline (speedup 1.0000x reference)
import functools

import jax
import jax.numpy as jnp
from jax import lax
from jax.experimental import pallas as pl
from jax.experimental.pallas import tpu as pltpu
from jax.experimental.pallas import tpu_sc as plsc


def _sc_geometry():
    try:
        info = plsc.get_sparse_core_info()
        return info.num_cores, info.num_subcores
    except Exception:
        return 2, 16


@functools.cache
def _build_gather(num_rows: int, dim: int, batch: int):
    NC, NS = _sc_geometry()
    NW = NC * NS
    assert batch % NW == 0
    b_per_w = batch // NW
    chunk = 512
    while b_per_w % chunk != 0:
        chunk //= 2
    n_chunks = b_per_w // chunk

    mesh = plsc.VectorSubcoreMesh(core_axis_name="c", subcore_axis_name="s")

    @functools.partial(
        pl.kernel,
        mesh=mesh,
        out_type=jax.ShapeDtypeStruct((batch, dim), jnp.float32),
        scratch_types=[
            pltpu.VMEM((chunk,), jnp.int32),
            pltpu.VMEM((chunk, dim), jnp.float32),
            pltpu.SemaphoreType.DMA,
        ],
    )
    def gather_kernel(table_hbm, idx_hbm, out_hbm, idx_v, rows_v, sem):
        wid = lax.axis_index("s") * NC + lax.axis_index("c")
        base = wid * b_per_w

        def body(i, carry):
            off = base + i * chunk
            pltpu.sync_copy(idx_hbm.at[pl.ds(off, chunk)], idx_v)
            pltpu.async_copy(table_hbm.at[idx_v], rows_v, sem).wait()
            pltpu.sync_copy(rows_v, out_hbm.at[pl.ds(off, chunk)])
            return carry

        lax.fori_loop(0, n_chunks, body, 0)

    return gather_kernel


def kernel(labels, embeddings):
    bsz, fields = labels.shape
    num_rows, dim = embeddings.shape
    flat = labels.reshape(bsz * fields)
    fn = _build_gather(num_rows, dim, bsz * fields)
    out = fn(embeddings, flat)
    return out.reshape(bsz, fields, dim)


# speedup vs baseline: 8.5258x; 8.5258x over previous
import functools

import jax
import jax.numpy as jnp
from jax import lax
from jax.experimental import pallas as pl
from jax.experimental.pallas import tpu as pltpu
from jax.experimental.pallas import tpu_sc as plsc


def _sc_geometry():
    try:
        info = plsc.get_sparse_core_info()
        return info.num_cores, info.num_subcores
    except Exception:
        return 2, 16


@functools.cache
def _build_gather(num_rows: int, dim: int, batch: int):
    NC, NS = _sc_geometry()
    NW = NC * NS
    assert batch % NW == 0
    b_per_w = batch // NW
    chunk = 512
    while b_per_w % chunk != 0:
        chunk //= 2
    n_chunks = b_per_w // chunk

    mesh = plsc.VectorSubcoreMesh(core_axis_name="c", subcore_axis_name="s")

    @functools.partial(
        pl.kernel,
        mesh=mesh,
        out_type=jax.ShapeDtypeStruct((batch, dim), jnp.float32),
        scratch_types=[
            pltpu.VMEM((chunk,), jnp.int32),
            pltpu.VMEM((chunk, dim), jnp.float32),
            pltpu.SemaphoreType.DMA,
        ],
        compiler_params=pltpu.CompilerParams(use_tc_tiling_on_sc=False),
    )
    def gather_kernel(table_hbm, idx_hbm, out_hbm, idx_v, rows_v, sem):
        wid = lax.axis_index("s") * NC + lax.axis_index("c")
        base = wid * b_per_w

        def body(i, carry):
            off = base + i * chunk
            pltpu.sync_copy(idx_hbm.at[pl.ds(off, chunk)], idx_v)
            pltpu.async_copy(table_hbm.at[idx_v], rows_v, sem).wait()
            pltpu.sync_copy(rows_v, out_hbm.at[pl.ds(off, chunk)])
            return carry

        lax.fori_loop(0, n_chunks, body, 0)

    return gather_kernel


def kernel(labels, embeddings):
    bsz, fields = labels.shape
    num_rows, dim = embeddings.shape
    flat = labels.reshape(bsz * fields)
    fn = _build_gather(num_rows, dim, bsz * fields)
    out = fn(embeddings, flat)
    return out.reshape(bsz, fields, dim)


# speedup vs baseline: 9.0834x; 1.0654x over previous
import functools

import jax
import jax.numpy as jnp
from jax import lax
from jax.experimental import pallas as pl
from jax.experimental.pallas import tpu as pltpu
from jax.experimental.pallas import tpu_sc as plsc


def _sc_geometry():
    try:
        info = plsc.get_sparse_core_info()
        return info.num_cores, info.num_subcores
    except Exception:
        return 2, 16


@functools.cache
def _build_gather(num_rows: int, dim: int, batch: int):
    NC, NS = _sc_geometry()
    NW = NC * NS
    assert batch % NW == 0
    b_per_w = batch // NW
    chunk = 512
    while b_per_w % chunk != 0:
        chunk //= 2
    n = b_per_w // chunk
    NB = 3
    assert n >= NB

    mesh = plsc.VectorSubcoreMesh(core_axis_name="c", subcore_axis_name="s")

    @functools.partial(
        pl.kernel,
        mesh=mesh,
        out_type=jax.ShapeDtypeStruct((batch, dim), jnp.float32),
        scratch_types=(
            [pltpu.VMEM((chunk,), jnp.int32)] * NB
            + [pltpu.VMEM((chunk, dim), jnp.float32)] * NB
            + [pltpu.SemaphoreType.DMA] * (3 * NB)
        ),
        compiler_params=pltpu.CompilerParams(use_tc_tiling_on_sc=False),
    )
    def gather_kernel(table_hbm, idx_hbm, out_hbm, *scratch):
        idx_v = scratch[0:NB]
        rows_v = scratch[NB:2 * NB]
        sems = scratch[2 * NB:]
        idx_sem = sems[0:NB]
        row_sem = sems[NB:2 * NB]
        out_sem = sems[2 * NB:3 * NB]
        wid = lax.axis_index("s") * NC + lax.axis_index("c")
        base = wid * b_per_w

        def idx_copy(i, b):
            return pltpu.make_async_copy(
                idx_hbm.at[pl.ds(base + i * chunk, chunk)],
                idx_v[b], idx_sem[b])

        def gather_copy(b):
            return pltpu.make_async_copy(
                table_hbm.at[idx_v[b]], rows_v[b], row_sem[b])

        def out_copy(i, b):
            return pltpu.make_async_copy(
                rows_v[b],
                out_hbm.at[pl.ds(base + i * chunk, chunk)], out_sem[b])

        for b in range(NB):
            idx_copy(b, b).start()

        n_groups = (n + 1 + NB - 1) // NB

        def group(g, carry):
            for b_pos in range(NB):
                i = g * NB + b_pos
                bj = (b_pos - 1) % NB

                @pl.when(i < n)
                def _():
                    @pl.when(i >= NB)
                    def _():
                        out_copy(i - NB, b_pos).wait()

                    idx_copy(i, b_pos).wait()
                    gather_copy(b_pos).start()

                @pl.when(jnp.logical_and(i >= 1, i <= n))
                def _():
                    gather_copy(bj).wait()
                    out_copy(i - 1, bj).start()

                    @pl.when(i - 1 + NB < n)
                    def _():
                        idx_copy(i - 1 + NB, bj).start()

            return carry

        lax.fori_loop(0, n_groups, group, 0)

        for j in range(n - NB, n):
            out_copy(j, j % NB).wait()

    return gather_kernel


def kernel(labels, embeddings):
    bsz, fields = labels.shape
    num_rows, dim = embeddings.shape
    flat = labels.reshape(bsz * fields)
    fn = _build_gather(num_rows, dim, bsz * fields)
    out = fn(embeddings, flat)
    return out.reshape(bsz, fields, dim)
